# Initial kernel scaffold; baseline (speedup 1.0000x reference)
import jax
import jax.numpy as jnp
from jax.experimental import pallas as pl


def kernel(x, edge_index, W1, att_src1, att_dst1, b1, W2, att_src2, att_dst2, b2):
    raise NotImplementedError("write your pallas kernel here")


# speedup vs baseline: 14.7634x; 14.7634x over previous
import functools

import jax
import jax.numpy as jnp
from jax import lax
from jax.experimental import pallas as pl
from jax.experimental.pallas import tpu as pltpu
from jax.experimental.pallas import tpu_sc as plsc

N = 10000
E = 320000
D = 128
NT = 32
CH = 128
NCH = 80
EPT = CH * NCH
E_PAD = EPT * NT
NP = 10112
RPT = NP // 16
BLK = 1000


def _front_body(x_ref, w_ref, as_ref, ad_ref, h_ref, aso_ref, ado_ref):
    h = jnp.dot(x_ref[...], w_ref[...], preferred_element_type=jnp.float32)
    h_ref[...] = h
    cdims = (((1,), (1,)), ((), ()))
    a_s = lax.dot_general(as_ref[...], h, cdims,
                          preferred_element_type=jnp.float32)
    a_d = lax.dot_general(ad_ref[...], h, cdims,
                          preferred_element_type=jnp.float32)
    aso_ref[...] = a_s.reshape((N,))
    ado_ref[...] = a_d.reshape((N,))


def _tc_front(x, w, att_s, att_d):
    return pl.pallas_call(
        _front_body,
        out_shape=[
            jax.ShapeDtypeStruct((N, D), jnp.float32),
            jax.ShapeDtypeStruct((N,), jnp.float32),
            jax.ShapeDtypeStruct((N,), jnp.float32),
        ],
    )(x, w, att_s, att_d)


def _combine_body(p_ref, den_ref, ones_ref, b_ref, out_ref):
    num = p_ref[0, :N, :] + p_ref[1, :N, :]
    den = lax.dot_general(
        den_ref[...], ones_ref[...],
        (((0,), (0,)), ((), ())), preferred_element_type=jnp.float32)
    out_ref[...] = jnp.maximum(num / (den[:N] + 1e-16) + b_ref[...], 0.0)


def _tc_combine(p, dens, b):
    ones = jnp.ones((NT, 1), jnp.float32)
    return pl.pallas_call(
        _combine_body,
        out_shape=jax.ShapeDtypeStruct((N, D), jnp.float32),
    )(p, dens, ones, b)


def _sc_edge_body(h_hbm, as_hbm, ad_hbm, src_hbm, dst_hbm, out_hbm, den_hbm,
                  acc, asrc, adst, srcb, dstb, exb, rows, denomb, sem):
    cid = lax.axis_index("c")
    sid = lax.axis_index("s")
    wid = sid * 2 + cid
    row0 = sid * RPT

    zeros16 = jnp.zeros((16,), jnp.float32)
    lanes = lax.iota(jnp.int32, 16)

    def zrow(i, carry):
        for k in range(D // 16):
            rows[i, pl.ds(16 * k, 16)] = zeros16
        return carry

    lax.fori_loop(0, CH, zrow, 0)

    def zden(i, carry):
        denomb[pl.ds(16 * i, 16)] = zeros16
        return carry

    lax.fori_loop(0, NP // 16, zden, 0)

    for off in range(0, RPT, CH):
        sz = min(CH, RPT - off)
        pltpu.sync_copy(rows.at[pl.ds(0, sz)], acc.at[pl.ds(row0 + off, sz)])
    plsc.subcore_barrier()

    pltpu.sync_copy(as_hbm, asrc)
    pltpu.sync_copy(ad_hbm, adst)

    def chunk(ci, carry):
        pltpu.sync_copy(src_hbm.at[wid, ci], srcb)
        pltpu.sync_copy(dst_hbm.at[wid, ci], dstb)
        cp = pltpu.async_copy(h_hbm.at[srcb], rows, sem)
        gbase = wid * EPT + ci * CH
        for j in range(CH // 16):
            si = srcb[pl.ds(16 * j, 16)]
            di = dstb[pl.ds(16 * j, 16)]
            t = plsc.load_gather(asrc, [si]) + plsc.load_gather(adst, [di])
            e = jnp.where(t > 0, t, t * 0.2)
            ex = jnp.exp(e)
            ex = jnp.where(gbase + 16 * j + lanes < E, ex, 0.0)
            exb[pl.ds(16 * j, 16)] = ex
            for l in range(16):
                plsc.addupdate_scatter(denomb, [di], ex, mask=lanes == l)
        cp.wait()

        def group(g, c2):
            exv = exb[pl.ds(g * 16, 16)]
            for j in range(16):
                scale = exv[j]
                i = g * 16 + j
                for k in range(D // 16):
                    rows[i, pl.ds(16 * k, 16)] = (
                        rows[i, pl.ds(16 * k, 16)] * scale)
            return c2

        lax.fori_loop(0, CH // 16, group, 0)
        pltpu.sync_copy(rows, acc.at[dstb], add=True)
        return carry

    lax.fori_loop(0, NCH, chunk, 0)
    plsc.subcore_barrier()

    for off in range(0, RPT, CH):
        sz = min(CH, RPT - off)
        pltpu.sync_copy(acc.at[pl.ds(row0 + off, sz)],
                        out_hbm.at[cid, pl.ds(row0 + off, sz)])
    pltpu.sync_copy(denomb, den_hbm.at[pl.ds(wid * NP, NP)])


_sc_edge = functools.partial(
    pl.kernel,
    out_type=[
        jax.ShapeDtypeStruct((2, NP, D), jnp.float32),
        jax.ShapeDtypeStruct((NT * NP,), jnp.float32),
    ],
    mesh=plsc.VectorSubcoreMesh(core_axis_name="c", subcore_axis_name="s"),
    compiler_params=pltpu.CompilerParams(needs_layout_passes=False),
    scratch_types=[
        pltpu.VMEM_SHARED((NP, D), jnp.float32),
        pltpu.VMEM((N,), jnp.float32),
        pltpu.VMEM((N,), jnp.float32),
        pltpu.VMEM((CH,), jnp.int32),
        pltpu.VMEM((CH,), jnp.int32),
        pltpu.VMEM((CH,), jnp.float32),
        pltpu.VMEM((CH, D), jnp.float32),
        pltpu.VMEM((NP,), jnp.float32),
        pltpu.SemaphoreType.DMA,
    ],
)(_sc_edge_body)


def _layer(x, w, att_s, att_d, b, src3, dst3):
    h, a_s, a_d = _tc_front(x, w, att_s, att_d)
    p, dens = _sc_edge(h, a_s, a_d, src3, dst3)
    return _tc_combine(p, dens.reshape(NT, NP), b)


@jax.jit
def kernel(x, edge_index, W1, att_src1, att_dst1, b1,
           W2, att_src2, att_dst2, b2):
    pad = jnp.zeros((E_PAD - E,), jnp.int32)
    src3 = jnp.concatenate(
        [edge_index[0].astype(jnp.int32), pad]).reshape(NT, NCH, CH)
    dst3 = jnp.concatenate(
        [edge_index[1].astype(jnp.int32), pad]).reshape(NT, NCH, CH)

    h1 = _layer(x, W1, att_src1[None, :], att_dst1[None, :], b1[None, :],
                src3, dst3)
    h2 = _layer(h1, W2, att_src2[None, :], att_dst2[None, :], b2[None, :],
                src3, dst3)
    return h2


# speedup vs baseline: 16.9517x; 1.1482x over previous
import functools

import jax
import jax.numpy as jnp
from jax import lax
from jax.experimental import pallas as pl
from jax.experimental.pallas import tpu as pltpu
from jax.experimental.pallas import tpu_sc as plsc

N = 10000
E = 320000
D = 128
NT = 32
CH = 64
NCH = 160
EPT = CH * NCH
E_PAD = EPT * NT
NP = 10112
RPT = NP // 16
BLK = 1000


def _front_body(x_ref, w_ref, as_ref, ad_ref, h_ref, aso_ref, ado_ref):
    h = jnp.dot(x_ref[...], w_ref[...], preferred_element_type=jnp.float32)
    h_ref[...] = h
    cdims = (((1,), (1,)), ((), ()))
    a_s = lax.dot_general(as_ref[...], h, cdims,
                          preferred_element_type=jnp.float32)
    a_d = lax.dot_general(ad_ref[...], h, cdims,
                          preferred_element_type=jnp.float32)
    aso_ref[...] = a_s.reshape((N,))
    ado_ref[...] = a_d.reshape((N,))


def _tc_front(x, w, att_s, att_d):
    return pl.pallas_call(
        _front_body,
        out_shape=[
            jax.ShapeDtypeStruct((N, D), jnp.float32),
            jax.ShapeDtypeStruct((N,), jnp.float32),
            jax.ShapeDtypeStruct((N,), jnp.float32),
        ],
    )(x, w, att_s, att_d)


def _combine_body(p_ref, den_ref, ones_ref, b_ref, out_ref):
    num = p_ref[0, :N, :] + p_ref[1, :N, :]
    den = lax.dot_general(
        den_ref[...], ones_ref[...],
        (((0,), (0,)), ((), ())), preferred_element_type=jnp.float32)
    out_ref[...] = jnp.maximum(num / (den[:N] + 1e-16) + b_ref[...], 0.0)


def _tc_combine(p, dens, b):
    ones = jnp.ones((NT, 1), jnp.float32)
    return pl.pallas_call(
        _combine_body,
        out_shape=jax.ShapeDtypeStruct((N, D), jnp.float32),
    )(p, dens, ones, b)


def _sc_edge_body(h_hbm, as_hbm, ad_hbm, src_hbm, dst_hbm, out_hbm, den_hbm,
                  acc, asrc, adst, srcb, dstb, exb, rows, denomb,
                  sem_g, sem_s):
    cid = lax.axis_index("c")
    sid = lax.axis_index("s")
    wid = sid * 2 + cid
    row0 = sid * RPT

    zeros16 = jnp.zeros((16,), jnp.float32)
    lanes = lax.iota(jnp.int32, 16)

    def zrow(i, carry):
        for k in range(D // 16):
            rows[0, i, pl.ds(16 * k, 16)] = zeros16
        return carry

    lax.fori_loop(0, CH, zrow, 0)

    def zden(i, carry):
        denomb[pl.ds(16 * i, 16)] = zeros16
        return carry

    lax.fori_loop(0, NP // 16, zden, 0)

    for off in range(0, RPT, CH):
        sz = min(CH, RPT - off)
        pltpu.sync_copy(rows.at[0, pl.ds(0, sz)],
                        acc.at[pl.ds(row0 + off, sz)])
    plsc.subcore_barrier()

    pltpu.sync_copy(as_hbm, asrc)
    pltpu.sync_copy(ad_hbm, adst)

    pltpu.sync_copy(src_hbm.at[wid, 0], srcb.at[0])
    pltpu.sync_copy(dst_hbm.at[wid, 0], dstb.at[0])
    pltpu.async_copy(h_hbm.at[srcb.at[0]], rows.at[0], sem_g)

    def chunk(ci, carry):
        cur = lax.rem(ci, 2)
        nxt = 1 - cur

        @pl.when(ci >= 1)
        def _():
            pltpu.make_async_copy(
                rows.at[nxt], acc.at[dstb.at[nxt]], sem_s).wait()

        @pl.when(ci + 1 < NCH)
        def _():
            pltpu.sync_copy(src_hbm.at[wid, ci + 1], srcb.at[nxt])
            pltpu.sync_copy(dst_hbm.at[wid, ci + 1], dstb.at[nxt])
            pltpu.async_copy(h_hbm.at[srcb.at[nxt]], rows.at[nxt], sem_g)

        gbase = wid * EPT + ci * CH
        for j in range(CH // 16):
            si = srcb[cur, pl.ds(16 * j, 16)]
            di = dstb[cur, pl.ds(16 * j, 16)]
            t = plsc.load_gather(asrc, [si]) + plsc.load_gather(adst, [di])
            e = jnp.where(t > 0, t, t * 0.2)
            ex = jnp.exp(e)
            ex = jnp.where(gbase + 16 * j + lanes < E, ex, 0.0)
            exb[cur, pl.ds(16 * j, 16)] = ex
            for l in range(16):
                plsc.addupdate_scatter(denomb, [di], ex, mask=lanes == l)

        pltpu.make_async_copy(
            h_hbm.at[srcb.at[cur]], rows.at[cur], sem_g).wait()

        @plsc.parallel_loop(0, CH // 16, unroll=2)
        def _(g):
            exv = exb[cur, pl.ds(g * 16, 16)]
            for j in range(16):
                scale = exv[j]
                i = g * 16 + j
                for k in range(D // 16):
                    rows[cur, i, pl.ds(16 * k, 16)] = (
                        rows[cur, i, pl.ds(16 * k, 16)] * scale)

        pltpu.async_copy(rows.at[cur], acc.at[dstb.at[cur]], sem_s, add=True)
        return carry

    lax.fori_loop(0, NCH, chunk, 0)
    last = lax.rem(NCH - 1, 2)
    pltpu.make_async_copy(
        rows.at[last], acc.at[dstb.at[last]], sem_s).wait()
    plsc.subcore_barrier()

    for off in range(0, RPT, 2 * CH):
        sz = min(2 * CH, RPT - off)
        pltpu.sync_copy(acc.at[pl.ds(row0 + off, sz)],
                        out_hbm.at[cid, pl.ds(row0 + off, sz)])
    pltpu.sync_copy(denomb, den_hbm.at[pl.ds(wid * NP, NP)])


_sc_edge = functools.partial(
    pl.kernel,
    out_type=[
        jax.ShapeDtypeStruct((2, NP, D), jnp.float32),
        jax.ShapeDtypeStruct((NT * NP,), jnp.float32),
    ],
    mesh=plsc.VectorSubcoreMesh(core_axis_name="c", subcore_axis_name="s"),
    compiler_params=pltpu.CompilerParams(needs_layout_passes=False),
    scratch_types=[
        pltpu.VMEM_SHARED((NP, D), jnp.float32),
        pltpu.VMEM((N,), jnp.float32),
        pltpu.VMEM((N,), jnp.float32),
        pltpu.VMEM((2, CH), jnp.int32),
        pltpu.VMEM((2, CH), jnp.int32),
        pltpu.VMEM((2, CH), jnp.float32),
        pltpu.VMEM((2, CH, D), jnp.float32),
        pltpu.VMEM((NP,), jnp.float32),
        pltpu.SemaphoreType.DMA,
        pltpu.SemaphoreType.DMA,
    ],
)(_sc_edge_body)


def _layer(x, w, att_s, att_d, b, src3, dst3):
    h, a_s, a_d = _tc_front(x, w, att_s, att_d)
    p, dens = _sc_edge(h, a_s, a_d, src3, dst3)
    return _tc_combine(p, dens.reshape(NT, NP), b)


@jax.jit
def kernel(x, edge_index, W1, att_src1, att_dst1, b1,
           W2, att_src2, att_dst2, b2):
    pad = jnp.zeros((E_PAD - E,), jnp.int32)
    src3 = jnp.concatenate(
        [edge_index[0].astype(jnp.int32), pad]).reshape(NT, NCH, CH)
    dst3 = jnp.concatenate(
        [edge_index[1].astype(jnp.int32), pad]).reshape(NT, NCH, CH)

    h1 = _layer(x, W1, att_src1[None, :], att_dst1[None, :], b1[None, :],
                src3, dst3)
    h2 = _layer(h1, W2, att_src2[None, :], att_dst2[None, :], b2[None, :],
                src3, dst3)
    return h2
